# baseline (speedup 1.0000x reference)
import jax
import jax.numpy as jnp
from jax.experimental import pallas as pl
from jax.experimental.pallas import tpu as pltpu

_CH = 368
_NFULL = 27
_TAIL = 64


def _gcn_kernel(x_ref, w_ref, b_ref, adj_hbm, out_ref,
                support_ref, buf0, buf1, tail_buf, sems):
    bufs = (buf0, buf1)

    def start(i):
        if i < _NFULL:
            pltpu.make_async_copy(
                adj_hbm.at[pl.ds(i * _CH, _CH), :], bufs[i % 2], sems.at[i % 2]
            ).start()
        elif i == _NFULL:
            pltpu.make_async_copy(
                adj_hbm.at[pl.ds(_NFULL * _CH, _TAIL), :], tail_buf, sems.at[2]
            ).start()

    def wait(i):
        if i < _NFULL:
            pltpu.make_async_copy(
                adj_hbm.at[pl.ds(i * _CH, _CH), :], bufs[i % 2], sems.at[i % 2]
            ).wait()
        else:
            pltpu.make_async_copy(
                adj_hbm.at[pl.ds(_NFULL * _CH, _TAIL), :], tail_buf, sems.at[2]
            ).wait()

    for i in range(2):
        start(i)
    start(_NFULL)

    support_ref[...] = jnp.dot(
        x_ref[...], w_ref[...], preferred_element_type=jnp.float32
    )

    def logsoftmax_store(o, row0, rows):
        m = jnp.max(o, axis=1, keepdims=True)
        e = o - m
        out_ref[pl.ds(row0, rows), :] = e - jnp.log(
            jnp.sum(jnp.exp(e), axis=1, keepdims=True)
        )

    for i in range(_NFULL):
        wait(i)
        o = (
            jnp.dot(bufs[i % 2][...], support_ref[...],
                    preferred_element_type=jnp.float32)
            + b_ref[...]
        )
        logsoftmax_store(o, i * _CH, _CH)
        if i + 2 < _NFULL:
            start(i + 2)

    wait(_NFULL)
    o = (
        jnp.dot(tail_buf[...], support_ref[...],
                preferred_element_type=jnp.float32)
        + b_ref[...]
    )
    logsoftmax_store(o, _NFULL * _CH, _TAIL)


@jax.jit
def kernel(x, adj, W, b):
    n, nfeat = x.shape
    nclass = W.shape[1]
    b2 = b.reshape(1, nclass)
    return pl.pallas_call(
        _gcn_kernel,
        in_specs=[
            pl.BlockSpec(memory_space=pltpu.MemorySpace.VMEM),
            pl.BlockSpec(memory_space=pltpu.MemorySpace.VMEM),
            pl.BlockSpec(memory_space=pltpu.MemorySpace.VMEM),
            pl.BlockSpec(memory_space=pl.ANY),
        ],
        out_specs=pl.BlockSpec(memory_space=pltpu.MemorySpace.VMEM),
        out_shape=jax.ShapeDtypeStruct((n, nclass), jnp.float32),
        scratch_shapes=[
            pltpu.VMEM((n, nclass), jnp.float32),
            pltpu.VMEM((_CH, n), jnp.float32),
            pltpu.VMEM((_CH, n), jnp.float32),
            pltpu.VMEM((_TAIL, n), jnp.float32),
            pltpu.SemaphoreType.DMA((3,)),
        ],
    )(x, W, b2, adj)


# speedup vs baseline: 1.0488x; 1.0488x over previous
import jax
import jax.numpy as jnp
from jax import lax
from jax.experimental import pallas as pl
from jax.experimental.pallas import tpu as pltpu

_CH = 368
_NFULL = 27
_TAIL = 64


def _gcn_kernel(x_ref, w_ref, b_ref, adj_hbm, out_ref,
                st_ref, buf0, buf1, buf2, tail_buf, sems):
    bufs = (buf0, buf1, buf2)

    def start(i):
        if i < _NFULL:
            pltpu.make_async_copy(
                adj_hbm.at[pl.ds(i * _CH, _CH), :], bufs[i % 3], sems.at[i % 3]
            ).start()
        elif i == _NFULL:
            pltpu.make_async_copy(
                adj_hbm.at[pl.ds(_NFULL * _CH, _TAIL), :], tail_buf, sems.at[3]
            ).start()

    def wait(i):
        if i < _NFULL:
            pltpu.make_async_copy(
                adj_hbm.at[pl.ds(i * _CH, _CH), :], bufs[i % 3], sems.at[i % 3]
            ).wait()
        else:
            pltpu.make_async_copy(
                adj_hbm.at[pl.ds(_NFULL * _CH, _TAIL), :], tail_buf, sems.at[3]
            ).wait()

    for i in range(3):
        start(i)
    start(_NFULL)

    st_ref[...] = lax.dot_general(
        w_ref[...], x_ref[...],
        (((0,), (1,)), ((), ())),
        preferred_element_type=jnp.float32,
    )

    def block_out(a_buf, row0, rows):
        o = lax.dot_general(
            a_buf[...], st_ref[...],
            (((1,), (1,)), ((), ())),
            preferred_element_type=jnp.float32,
        ) + b_ref[...]
        m = jnp.max(o, axis=1, keepdims=True)
        e = o - m
        out_ref[pl.ds(row0, rows), :] = e - jnp.log(
            jnp.sum(jnp.exp(e), axis=1, keepdims=True)
        )

    for i in range(_NFULL):
        wait(i)
        block_out(bufs[i % 3], i * _CH, _CH)
        if i + 3 < _NFULL:
            start(i + 3)

    wait(_NFULL)
    block_out(tail_buf, _NFULL * _CH, _TAIL)


@jax.jit
def kernel(x, adj, W, b):
    n, nfeat = x.shape
    nclass = W.shape[1]
    b2 = b.reshape(1, nclass)
    return pl.pallas_call(
        _gcn_kernel,
        in_specs=[
            pl.BlockSpec(memory_space=pltpu.MemorySpace.VMEM),
            pl.BlockSpec(memory_space=pltpu.MemorySpace.VMEM),
            pl.BlockSpec(memory_space=pltpu.MemorySpace.VMEM),
            pl.BlockSpec(memory_space=pl.ANY),
        ],
        out_specs=pl.BlockSpec(memory_space=pltpu.MemorySpace.VMEM),
        out_shape=jax.ShapeDtypeStruct((n, nclass), jnp.float32),
        scratch_shapes=[
            pltpu.VMEM((nclass, n), jnp.float32),
            pltpu.VMEM((_CH, n), jnp.float32),
            pltpu.VMEM((_CH, n), jnp.float32),
            pltpu.VMEM((_CH, n), jnp.float32),
            pltpu.VMEM((_TAIL, n), jnp.float32),
            pltpu.SemaphoreType.DMA((4,)),
        ],
    )(x, W, b2, adj)


# speedup vs baseline: 1.0963x; 1.0453x over previous
import jax
import jax.numpy as jnp
from jax import lax
from jax.experimental import pallas as pl
from jax.experimental.pallas import tpu as pltpu

_BM = 400


def _gcn_block_kernel(x_ref, w_ref, b_ref, adj_ref, out_ref, st_ref):
    @pl.when(pl.program_id(0) == 0)
    def _compute_support():
        st_ref[...] = lax.dot_general(
            w_ref[...], x_ref[...],
            (((0,), (1,)), ((), ())),
            preferred_element_type=jnp.float32,
        )

    o = lax.dot_general(
        adj_ref[...], st_ref[...],
        (((1,), (1,)), ((), ())),
        preferred_element_type=jnp.float32,
    ) + b_ref[...]
    m = jnp.max(o, axis=1, keepdims=True)
    e = o - m
    out_ref[...] = e - jnp.log(jnp.sum(jnp.exp(e), axis=1, keepdims=True))


@jax.jit
def kernel(x, adj, W, b):
    n, nfeat = x.shape
    nclass = W.shape[1]
    b2 = b.reshape(1, nclass)
    return pl.pallas_call(
        _gcn_block_kernel,
        grid=(pl.cdiv(n, _BM),),
        in_specs=[
            pl.BlockSpec((n, nfeat), lambda i: (0, 0)),
            pl.BlockSpec((nfeat, nclass), lambda i: (0, 0)),
            pl.BlockSpec((1, nclass), lambda i: (0, 0)),
            pl.BlockSpec((_BM, n), lambda i: (i, 0)),
        ],
        out_specs=pl.BlockSpec((_BM, nclass), lambda i: (i, 0)),
        out_shape=jax.ShapeDtypeStruct((n, nclass), jnp.float32),
        scratch_shapes=[pltpu.VMEM((nclass, n), jnp.float32)],
        compiler_params=pltpu.CompilerParams(
            dimension_semantics=("arbitrary",),
        ),
    )(x, W, b2, adj)
